# baseline (speedup 1.0000x reference)
import jax
import jax.numpy as jnp
from jax.experimental import pallas as pl
from jax.experimental.pallas import tpu as pltpu

_NH, _HD = 16, 128
_CHUNK, _TOPK = 256, 4
_THETA = 10000.0
_SCALE = _HD ** -0.5


def _dot_t(a, b):
    return jax.lax.dot_general(a, b, (((1,), (1,)), ((), ())),
                               preferred_element_type=jnp.float32)


def _qkv_kernel(h_ref, w_ref, cos_ref, sin_ref, o_ref):
    j = pl.program_id(1)
    x = _dot_t(h_ref[...], w_ref[...])
    half = _HD // 2
    x1 = x[:, :half]
    x2 = x[:, half:]
    cos = cos_ref[...]
    sin = sin_ref[...]
    roped = jnp.concatenate([x1 * cos - x2 * sin, x2 * cos + x1 * sin], axis=1)
    o_ref[...] = jnp.where(j < 2 * _NH, roped, x)


def _attn_kernel(q_ref, k_ref, v_ref, o_ref, *, nchunks):
    qt = pl.program_id(2)
    q = q_ref[...]
    k = k_ref[...]

    kmean = jnp.mean(k.reshape(nchunks, _CHUNK, _HD), axis=1)
    gates = _dot_t(q, kmean) * _SCALE
    cidx = jax.lax.broadcasted_iota(jnp.int32, gates.shape, 1)
    g = jnp.where(cidx == qt, 1e9, gates)
    g = jnp.where(cidx > qt, -1e9, g)
    sel_cols = []
    for c in range(nchunks):
        gc = g[:, c:c + 1]
        beats = (g > gc) | ((g == gc) & (cidx < c))
        rank = jnp.sum(beats.astype(jnp.int32), axis=1, keepdims=True)
        sel_cols.append((rank < _TOPK).astype(jnp.float32))
    sel = jnp.concatenate(sel_cols, axis=1)

    rowi = jax.lax.broadcasted_iota(jnp.int32, (_CHUNK, _CHUNK), 0)
    coli = jax.lax.broadcasted_iota(jnp.int32, (_CHUNK, _CHUNK), 1)
    tri = rowi >= coli
    qb = q.astype(jnp.bfloat16)

    def body(c, carry):
        acc, m, l = carry
        kc = k_ref[pl.ds(c * _CHUNK, _CHUNK), :].astype(jnp.bfloat16)
        vc = v_ref[pl.ds(c * _CHUNK, _CHUNK), :].astype(jnp.bfloat16)
        s = _dot_t(qb, kc) * _SCALE
        selc = jnp.sum(jnp.where(cidx == c, sel, 0.0),
                       axis=1, keepdims=True) > 0.5
        msk = selc & (tri | (c != qt))
        s = jnp.where(msk, s, -1e30)
        mnew = jnp.maximum(m, jnp.max(s, axis=1, keepdims=True))
        p = jnp.exp(s - mnew)
        p = jnp.where(msk, p, 0.0)
        alpha = jnp.exp(m - mnew)
        lnew = l * alpha + jnp.sum(p, axis=1, keepdims=True)
        accnew = acc * alpha + jnp.dot(p.astype(jnp.bfloat16), vc,
                                       preferred_element_type=jnp.float32)
        return accnew, mnew, lnew

    acc0 = jnp.zeros((_CHUNK, _HD), jnp.float32)
    m0 = jnp.full((_CHUNK, 1), -1e30, jnp.float32)
    l0 = jnp.zeros((_CHUNK, 1), jnp.float32)
    acc, _, l = jax.lax.fori_loop(0, qt + 1, body, (acc0, m0, l0))
    o_ref[...] = acc / l


def _proj_kernel(x_ref, w_ref, o_ref):
    o_ref[...] = jax.lax.dot_general(
        x_ref[...].astype(jnp.bfloat16), w_ref[...].astype(jnp.bfloat16),
        (((1,), (1,)), ((), ())), preferred_element_type=jnp.float32)


def kernel(hidden_states, positions, Wqkv, Wo):
    b, s, hid = hidden_states.shape
    nchunks = s // _CHUNK
    rows = b * s
    hs = hidden_states.reshape(rows, hid)

    inv = 1.0 / (_THETA ** (jnp.arange(0, _HD, 2, dtype=jnp.float32) / _HD))
    f = positions.astype(jnp.float32)[:, None] * inv[None, :]
    cos = jnp.cos(f)
    sin = jnp.sin(f)

    qkv = pl.pallas_call(
        _qkv_kernel,
        grid=(b, 3 * _NH),
        in_specs=[
            pl.BlockSpec((s, hid), lambda i, j: (i, 0)),
            pl.BlockSpec((_HD, hid), lambda i, j: (j, 0)),
            pl.BlockSpec((s, _HD // 2), lambda i, j: (0, 0)),
            pl.BlockSpec((s, _HD // 2), lambda i, j: (0, 0)),
        ],
        out_specs=pl.BlockSpec((s, _HD), lambda i, j: (i, j)),
        out_shape=jax.ShapeDtypeStruct((rows, 3 * _NH * _HD), jnp.float32),
    )(hs, Wqkv, cos, sin)

    import functools
    attn = pl.pallas_call(
        functools.partial(_attn_kernel, nchunks=nchunks),
        grid=(b, _NH, nchunks),
        in_specs=[
            pl.BlockSpec((_CHUNK, _HD), lambda bi, h, qt: (bi * nchunks + qt, h)),
            pl.BlockSpec((s, _HD), lambda bi, h, qt: (bi, _NH + h)),
            pl.BlockSpec((s, _HD), lambda bi, h, qt: (bi, 2 * _NH + h)),
        ],
        out_specs=pl.BlockSpec((_CHUNK, _HD),
                               lambda bi, h, qt: (bi * nchunks + qt, h)),
        out_shape=jax.ShapeDtypeStruct((rows, _NH * _HD), jnp.float32),
    )(qkv, qkv, qkv)

    return attn.reshape(b, s, hid)
    ocols = 512
    out = pl.pallas_call(
        _proj_kernel,
        grid=(b, hid // ocols),
        in_specs=[
            pl.BlockSpec((s, _NH * _HD), lambda i, j: (i, 0)),
            pl.BlockSpec((ocols, _NH * _HD), lambda i, j: (j, 0)),
        ],
        out_specs=pl.BlockSpec((s, ocols), lambda i, j: (i, j)),
        out_shape=jax.ShapeDtypeStruct((rows, hid), jnp.float32),
    )(attn, Wo)
    return out.reshape(b, s, hid)


# speedup vs baseline: 2.1062x; 2.1062x over previous
import functools

import jax
import jax.numpy as jnp
from jax.experimental import pallas as pl
from jax.experimental.pallas import tpu as pltpu

_NH, _HD = 16, 128
_CHUNK, _TOPK = 256, 4
_THETA = 10000.0
_SCALE = _HD ** -0.5
_NEG = -2e30


def _dot_t(a, b):
    return jax.lax.dot_general(a, b, (((1,), (1,)), ((), ())),
                               preferred_element_type=jnp.float32)


def _qk_kernel(h_ref, w_ref, cos_ref, sin_ref, o_ref):
    x = _dot_t(h_ref[...], w_ref[...])
    half = _HD // 2
    x1 = x[:, :half]
    x2 = x[:, half:]
    cos = cos_ref[...]
    sin = sin_ref[...]
    o_ref[...] = jnp.concatenate(
        [x1 * cos - x2 * sin, x2 * cos + x1 * sin], axis=1)


def _v_kernel(h_ref, w_ref, o_ref):
    o_ref[...] = jax.lax.dot_general(
        h_ref[...].astype(jnp.bfloat16), w_ref[...].astype(jnp.bfloat16),
        (((1,), (1,)), ((), ())),
        preferred_element_type=jnp.float32).astype(jnp.bfloat16)


def _attn_kernel(q_ref, k_ref, v_ref, o_ref, *, nchunks):
    c_sz = _CHUNK
    kms = [jnp.mean(k_ref[c * c_sz:(c + 1) * c_sz, :], axis=0, keepdims=True)
           for c in range(nchunks)]
    kmean = jnp.concatenate(kms, axis=0)

    rowc = jax.lax.broadcasted_iota(jnp.int32, (nchunks, c_sz), 0)
    rowi = jax.lax.broadcasted_iota(jnp.int32, (c_sz, c_sz), 0)
    coli = jax.lax.broadcasted_iota(jnp.int32, (c_sz, c_sz), 1)
    tri = rowi >= coli

    for qt in range(nchunks):
        qtile = q_ref[qt * c_sz:(qt + 1) * c_sz, :]
        gt = _dot_t(kmean, qtile) * _SCALE
        g = jnp.where(rowc == qt, 1e9, gt)
        g = jnp.where(rowc > qt, -1e9, g)
        rank = jnp.zeros((nchunks, c_sz), jnp.float32)
        for d in range(1, nchunks):
            gs = jnp.concatenate([g[d:], g[:d]], axis=0)
            tie_lower = (rowc + d) >= nchunks
            beats = (gs > g) | ((gs == g) & tie_lower)
            rank = rank + beats.astype(jnp.float32)
        bias_t = jnp.where(rank < _TOPK, 0.0, _NEG)
        bias = bias_t.T

        qb = (qtile * _SCALE).astype(jnp.bfloat16)
        acc = jnp.zeros((c_sz, _HD), jnp.float32)
        m = jnp.full((c_sz, 1), -1e30, jnp.float32)
        l = jnp.zeros((c_sz, 1), jnp.float32)
        for c in range(qt + 1):
            kc = k_ref[c * c_sz:(c + 1) * c_sz, :].astype(jnp.bfloat16)
            s = _dot_t(qb, kc)
            if c == qt:
                s = jnp.where(tri, s, _NEG)
            else:
                s = s + bias[:, c:c + 1]
            mnew = jnp.maximum(m, jnp.max(s, axis=1, keepdims=True))
            alpha = jnp.exp(m - mnew)
            p = jnp.exp(s - mnew)
            l = l * alpha + jnp.sum(p, axis=1, keepdims=True)
            vc = v_ref[c * c_sz:(c + 1) * c_sz, :]
            acc = acc * alpha + jnp.dot(p.astype(jnp.bfloat16), vc,
                                        preferred_element_type=jnp.float32)
            m = mnew
        o_ref[qt * c_sz:(qt + 1) * c_sz, :] = (acc / l).astype(jnp.bfloat16)


def _proj_kernel(x_ref, w_ref, o_ref):
    o_ref[...] = jax.lax.dot_general(
        x_ref[...], w_ref[...].astype(jnp.bfloat16),
        (((1,), (1,)), ((), ())), preferred_element_type=jnp.float32)


def kernel(hidden_states, positions, Wqkv, Wo):
    b, s, hid = hidden_states.shape
    nchunks = s // _CHUNK
    rows = b * s
    hs = hidden_states.reshape(rows, hid)

    inv = 1.0 / (_THETA ** (jnp.arange(0, _HD, 2, dtype=jnp.float32) / _HD))
    f = positions.astype(jnp.float32)[:, None] * inv[None, :]
    cos = jnp.cos(f)
    sin = jnp.sin(f)

    params = pltpu.CompilerParams(
        dimension_semantics=("parallel", "parallel"))

    qk = pl.pallas_call(
        _qk_kernel,
        grid=(b, 2 * _NH),
        in_specs=[
            pl.BlockSpec((s, hid), lambda i, j: (i, 0)),
            pl.BlockSpec((_HD, hid), lambda i, j: (j, 0)),
            pl.BlockSpec((s, _HD // 2), lambda i, j: (0, 0)),
            pl.BlockSpec((s, _HD // 2), lambda i, j: (0, 0)),
        ],
        out_specs=pl.BlockSpec((s, _HD), lambda i, j: (i, j)),
        out_shape=jax.ShapeDtypeStruct((rows, 2 * _NH * _HD), jnp.float32),
        compiler_params=params,
    )(hs, Wqkv, cos, sin)

    v = pl.pallas_call(
        _v_kernel,
        grid=(b, _NH),
        in_specs=[
            pl.BlockSpec((s, hid), lambda i, j: (i, 0)),
            pl.BlockSpec((_HD, hid), lambda i, j: (2 * _NH + j, 0)),
        ],
        out_specs=pl.BlockSpec((s, _HD), lambda i, j: (i, j)),
        out_shape=jax.ShapeDtypeStruct((rows, _NH * _HD), jnp.bfloat16),
        compiler_params=params,
    )(hs, Wqkv)

    attn = pl.pallas_call(
        functools.partial(_attn_kernel, nchunks=nchunks),
        grid=(b, _NH),
        in_specs=[
            pl.BlockSpec((s, _HD), lambda bi, h: (bi, h)),
            pl.BlockSpec((s, _HD), lambda bi, h: (bi, _NH + h)),
            pl.BlockSpec((s, _HD), lambda bi, h: (bi, h)),
        ],
        out_specs=pl.BlockSpec((s, _HD), lambda bi, h: (bi, h)),
        out_shape=jax.ShapeDtypeStruct((rows, _NH * _HD), jnp.bfloat16),
        compiler_params=params,
    )(qk, qk, v)

    ocols = 512
    out = pl.pallas_call(
        _proj_kernel,
        grid=(b, hid // ocols),
        in_specs=[
            pl.BlockSpec((s, _NH * _HD), lambda i, j: (i, 0)),
            pl.BlockSpec((ocols, _NH * _HD), lambda i, j: (j, 0)),
        ],
        out_specs=pl.BlockSpec((s, ocols), lambda i, j: (i, j)),
        out_shape=jax.ShapeDtypeStruct((rows, hid), jnp.float32),
        compiler_params=params,
    )(attn, Wo)
    return out.reshape(b, s, hid)


# speedup vs baseline: 2.2229x; 1.0554x over previous
import functools

import jax
import jax.numpy as jnp
from jax.experimental import pallas as pl
from jax.experimental.pallas import tpu as pltpu

_NH, _HD = 16, 128
_CHUNK, _TOPK = 256, 4
_THETA = 10000.0
_SCALE = _HD ** -0.5
_NEG = -2e30


def _dot_t(a, b):
    return jax.lax.dot_general(a, b, (((1,), (1,)), ((), ())),
                               preferred_element_type=jnp.float32)


def _rope_head(xh, cos, sin):
    half = _HD // 2
    x1 = xh[:, :half]
    x2 = xh[:, half:]
    return jnp.concatenate([x1 * cos - x2 * sin, x2 * cos + x1 * sin], axis=1)


def _qk_kernel(h_ref, w_ref, cos_ref, sin_ref, o_ref):
    x = _dot_t(h_ref[...], w_ref[...])
    cos = cos_ref[...]
    sin = sin_ref[...]
    nheads = x.shape[1] // _HD
    for hh in range(nheads):
        xh = x[:, hh * _HD:(hh + 1) * _HD]
        o_ref[:, hh * _HD:(hh + 1) * _HD] = _rope_head(xh, cos, sin)


def _v_kernel(h_ref, w_ref, o_ref):
    o_ref[...] = jax.lax.dot_general(
        h_ref[...].astype(jnp.bfloat16), w_ref[...].astype(jnp.bfloat16),
        (((1,), (1,)), ((), ())),
        preferred_element_type=jnp.float32).astype(jnp.bfloat16)


def _attn_kernel(q_ref, k_ref, v_ref, o_ref, *, nchunks):
    c_sz = _CHUNK
    kms = [jnp.mean(k_ref[c * c_sz:(c + 1) * c_sz, :], axis=0, keepdims=True)
           for c in range(nchunks)]
    kmean = jnp.concatenate(kms, axis=0)

    rowc = jax.lax.broadcasted_iota(jnp.int32, (nchunks, c_sz), 0)
    rowk = jax.lax.broadcasted_iota(jnp.int32, (c_sz, c_sz), 0)
    colq = jax.lax.broadcasted_iota(jnp.int32, (c_sz, c_sz), 1)
    tri_t = colq >= rowk

    for qt in range(nchunks):
        qtile = q_ref[qt * c_sz:(qt + 1) * c_sz, :]
        gt = _dot_t(kmean, qtile) * _SCALE
        g = jnp.where(rowc == qt, 1e9, gt)
        g = jnp.where(rowc > qt, -1e9, g)
        rank = jnp.zeros((nchunks, c_sz), jnp.float32)
        for d in range(1, nchunks):
            gs = jnp.concatenate([g[d:], g[:d]], axis=0)
            tie_lower = (rowc + d) >= nchunks
            beats = (gs > g) | ((gs == g) & tie_lower)
            rank = rank + beats.astype(jnp.float32)
        bias_t = jnp.where(rank < _TOPK, 0.0, _NEG)

        qb = (qtile * _SCALE).astype(jnp.bfloat16)
        acc = jnp.zeros((_HD, c_sz), jnp.float32)
        m = jnp.full((1, c_sz), -1e30, jnp.float32)
        l = jnp.zeros((1, c_sz), jnp.float32)
        for c in range(qt + 1):
            kc = k_ref[c * c_sz:(c + 1) * c_sz, :].astype(jnp.bfloat16)
            s = _dot_t(kc, qb)
            if c == qt:
                s = jnp.where(tri_t, s, _NEG)
            else:
                s = s + bias_t[c:c + 1, :]
            mnew = jnp.maximum(m, jnp.max(s, axis=0, keepdims=True))
            alpha = jnp.exp(m - mnew)
            p = jnp.exp(s - mnew)
            l = l * alpha + jnp.sum(p, axis=0, keepdims=True)
            vc = v_ref[c * c_sz:(c + 1) * c_sz, :]
            pv = jax.lax.dot_general(vc, p.astype(jnp.bfloat16),
                                     (((0,), (0,)), ((), ())),
                                     preferred_element_type=jnp.float32)
            acc = acc * alpha + pv
            m = mnew
        o_ref[0, :, qt * c_sz:(qt + 1) * c_sz] = (
            acc * (1.0 / l)).astype(jnp.bfloat16)


def _proj_kernel(x_ref, w_ref, o_ref):
    o_ref[...] = jax.lax.dot_general(
        x_ref[0], w_ref[...].astype(jnp.bfloat16),
        (((0,), (1,)), ((), ())), preferred_element_type=jnp.float32)


def kernel(hidden_states, positions, Wqkv, Wo):
    b, s, hid = hidden_states.shape
    nchunks = s // _CHUNK
    rows = b * s
    hs = hidden_states.reshape(rows, hid)
    qk_heads = 2 * _NH

    inv = 1.0 / (_THETA ** (jnp.arange(0, _HD, 2, dtype=jnp.float32) / _HD))
    f = positions.astype(jnp.float32)[:, None] * inv[None, :]
    cos = jnp.cos(f)
    sin = jnp.sin(f)

    params = pltpu.CompilerParams(
        dimension_semantics=("parallel", "parallel"))

    hpt = 4
    ncol = hpt * _HD
    qk = pl.pallas_call(
        _qk_kernel,
        grid=(b, qk_heads // hpt),
        in_specs=[
            pl.BlockSpec((s, hid), lambda i, j: (i, 0)),
            pl.BlockSpec((ncol, hid), lambda i, j: (j, 0)),
            pl.BlockSpec((s, _HD // 2), lambda i, j: (0, 0)),
            pl.BlockSpec((s, _HD // 2), lambda i, j: (0, 0)),
        ],
        out_specs=pl.BlockSpec((s, ncol), lambda i, j: (i, j)),
        out_shape=jax.ShapeDtypeStruct((rows, qk_heads * _HD), jnp.float32),
        compiler_params=params,
    )(hs, Wqkv, cos, sin)

    v = pl.pallas_call(
        _v_kernel,
        grid=(b, _NH // hpt),
        in_specs=[
            pl.BlockSpec((s, hid), lambda i, j: (i, 0)),
            pl.BlockSpec((ncol, hid), lambda i, j: (qk_heads // hpt + j, 0)),
        ],
        out_specs=pl.BlockSpec((s, ncol), lambda i, j: (i, j)),
        out_shape=jax.ShapeDtypeStruct((rows, _NH * _HD), jnp.bfloat16),
        compiler_params=params,
    )(hs, Wqkv)

    attn_t = pl.pallas_call(
        functools.partial(_attn_kernel, nchunks=nchunks),
        grid=(b, _NH),
        in_specs=[
            pl.BlockSpec((s, _HD), lambda bi, h: (bi, h)),
            pl.BlockSpec((s, _HD), lambda bi, h: (bi, _NH + h)),
            pl.BlockSpec((s, _HD), lambda bi, h: (bi, h)),
        ],
        out_specs=pl.BlockSpec((1, _HD, s), lambda bi, h: (bi, h, 0)),
        out_shape=jax.ShapeDtypeStruct((b, _NH * _HD, s), jnp.bfloat16),
        compiler_params=params,
    )(qk, qk, v)

    ocols = 512
    out = pl.pallas_call(
        _proj_kernel,
        grid=(b, hid // ocols),
        in_specs=[
            pl.BlockSpec((1, _NH * _HD, s), lambda i, j: (i, 0, 0)),
            pl.BlockSpec((ocols, _NH * _HD), lambda i, j: (j, 0)),
        ],
        out_specs=pl.BlockSpec((s, ocols), lambda i, j: (i, j)),
        out_shape=jax.ShapeDtypeStruct((rows, hid), jnp.float32),
        compiler_params=params,
    )(attn_t, Wo)
    return out.reshape(b, s, hid)


# speedup vs baseline: 2.6190x; 1.1782x over previous
import functools

import jax
import jax.numpy as jnp
from jax.experimental import pallas as pl
from jax.experimental.pallas import tpu as pltpu

_NH, _HD = 16, 128
_CHUNK, _TOPK = 256, 4
_THETA = 10000.0
_SCALE = _HD ** -0.5
_NEG = -2e30


def _dot_t(a, b):
    return jax.lax.dot_general(a, b, (((1,), (1,)), ((), ())),
                               preferred_element_type=jnp.float32)


def _attn_kernel(h_ref, wq_ref, wk_ref, wv_ref, cc_ref, sc_ref, o_ref,
                 q_ref, k_ref, v_ref, *, nchunks):
    c_sz = _CHUNK
    nh = wq_ref.shape[0] // _HD
    hsls = [slice(hh * _HD, (hh + 1) * _HD) for hh in range(nh)]

    h = h_ref[...]
    cc = cc_ref[...]
    sc = sc_ref[...]
    q2 = _dot_t(h, wq_ref[...])
    k2 = _dot_t(h, wk_ref[...])
    for hsl in hsls:
        xq = q2[:, hsl]
        q_ref[:, hsl] = xq * cc + pltpu.roll(xq, _HD // 2, 1) * sc
        xk = k2[:, hsl]
        k_ref[:, hsl] = xk * cc + pltpu.roll(xk, _HD // 2, 1) * sc
    v_ref[...] = jax.lax.dot_general(
        h.astype(jnp.bfloat16), wv_ref[...].astype(jnp.bfloat16),
        (((1,), (1,)), ((), ())),
        preferred_element_type=jnp.float32).astype(jnp.bfloat16)

    rowc = jax.lax.broadcasted_iota(jnp.int32, (nchunks, c_sz), 0)
    rowk = jax.lax.broadcasted_iota(jnp.int32, (c_sz, c_sz), 0)
    colq = jax.lax.broadcasted_iota(jnp.int32, (c_sz, c_sz), 1)
    tri_t = colq >= rowk
    ones_hd = jnp.ones((1, _HD), jnp.float32)
    ones_c = jnp.ones((1, c_sz), jnp.bfloat16)

    for hh, hsl in enumerate(hsls):
        kms, kmax_pre = [], []
        for c in range(nchunks):
            kc = k_ref[c * c_sz:(c + 1) * c_sz, hsl]
            kms.append(jnp.mean(kc, axis=0, keepdims=True))
            kn2 = jax.lax.dot_general(ones_hd, kc * kc, (((1,), (1,)), ((), ())),
                                      preferred_element_type=jnp.float32)
            kn = jnp.max(jnp.sqrt(kn2))
            kmax_pre.append(kn if c == 0 else jnp.maximum(kmax_pre[-1], kn))
        kmean = jnp.concatenate(kms, axis=0)

        for qt in range(nchunks):
            qtile = q_ref[qt * c_sz:(qt + 1) * c_sz, hsl]
            gt = _dot_t(kmean, qtile) * _SCALE
            g = jnp.where(rowc == qt, 1e9, gt)
            g = jnp.where(rowc > qt, -1e9, g)
            rank = jnp.zeros((nchunks, c_sz), jnp.float32)
            for d in range(1, nchunks):
                gs = jnp.concatenate([g[d:], g[:d]], axis=0)
                tie_lower = (rowc + d) >= nchunks
                beats = (gs > g) | ((gs == g) & tie_lower)
                rank = rank + beats.astype(jnp.float32)
            bias_t = jnp.where(rank < _TOPK, 0.0, _NEG)

            qn2 = jax.lax.dot_general(ones_hd, qtile * qtile,
                                      (((1,), (1,)), ((), ())),
                                      preferred_element_type=jnp.float32)
            bnd = jnp.sqrt(qn2) * (kmax_pre[qt] * _SCALE)

            qb = (qtile * _SCALE).astype(jnp.bfloat16)
            acc = jnp.zeros((_HD, c_sz), jnp.float32)
            l = jnp.zeros((1, c_sz), jnp.float32)
            for c in range(qt + 1):
                kc = k_ref[c * c_sz:(c + 1) * c_sz, hsl].astype(jnp.bfloat16)
                s = _dot_t(kc, qb)
                adj = (-bnd) if c == qt else (bias_t[c:c + 1, :] - bnd)
                p = jnp.exp(s + adj)
                if c == qt:
                    p = jnp.where(tri_t, p, 0.0)
                pb = p.astype(jnp.bfloat16)
                l = l + jax.lax.dot_general(ones_c, pb, (((1,), (0,)), ((), ())),
                                            preferred_element_type=jnp.float32)
                vc = v_ref[c * c_sz:(c + 1) * c_sz, hsl]
                acc = acc + jax.lax.dot_general(
                    vc, pb, (((0,), (0,)), ((), ())),
                    preferred_element_type=jnp.float32)
            o_ref[0, hsl, qt * c_sz:(qt + 1) * c_sz] = (
                acc * (1.0 / l)).astype(jnp.bfloat16)


def _proj_kernel(x_ref, w_ref, o_ref):
    o_ref[...] = jax.lax.dot_general(
        x_ref[0], w_ref[...].astype(jnp.bfloat16),
        (((0,), (1,)), ((), ())), preferred_element_type=jnp.float32)


def kernel(hidden_states, positions, Wqkv, Wo):
    b, s, hid = hidden_states.shape
    nchunks = s // _CHUNK
    rows = b * s
    hs = hidden_states.reshape(rows, hid)

    inv = 1.0 / (_THETA ** (jnp.arange(0, _HD, 2, dtype=jnp.float32) / _HD))
    f = positions.astype(jnp.float32)[:, None] * inv[None, :]
    cos = jnp.cos(f)
    sin = jnp.sin(f)
    cc = jnp.concatenate([cos, cos], axis=1)
    sc = jnp.concatenate([-sin, sin], axis=1)

    params = pltpu.CompilerParams(
        dimension_semantics=("parallel", "parallel"))

    hpg = 2
    npairs = _NH // hpg
    wcol = hpg * _HD
    attn_t = pl.pallas_call(
        functools.partial(_attn_kernel, nchunks=nchunks),
        grid=(b, npairs),
        in_specs=[
            pl.BlockSpec((s, hid), lambda bi, h: (bi, 0)),
            pl.BlockSpec((wcol, hid), lambda bi, h: (h, 0)),
            pl.BlockSpec((wcol, hid), lambda bi, h: (_NH // 2 + h, 0)),
            pl.BlockSpec((wcol, hid), lambda bi, h: (_NH + h, 0)),
            pl.BlockSpec((s, _HD), lambda bi, h: (0, 0)),
            pl.BlockSpec((s, _HD), lambda bi, h: (0, 0)),
        ],
        out_specs=pl.BlockSpec((1, wcol, s), lambda bi, h: (bi, h, 0)),
        out_shape=jax.ShapeDtypeStruct((b, _NH * _HD, s), jnp.bfloat16),
        scratch_shapes=[
            pltpu.VMEM((s, wcol), jnp.float32),
            pltpu.VMEM((s, wcol), jnp.float32),
            pltpu.VMEM((s, wcol), jnp.bfloat16),
        ],
        compiler_params=params,
    )(hs, Wqkv, Wqkv, Wqkv, cc, sc)

    ocols = 512
    out = pl.pallas_call(
        _proj_kernel,
        grid=(b, hid // ocols),
        in_specs=[
            pl.BlockSpec((1, _NH * _HD, s), lambda i, j: (i, 0, 0)),
            pl.BlockSpec((ocols, _NH * _HD), lambda i, j: (j, 0)),
        ],
        out_specs=pl.BlockSpec((s, ocols), lambda i, j: (i, j)),
        out_shape=jax.ShapeDtypeStruct((rows, hid), jnp.float32),
        compiler_params=params,
    )(attn_t, Wo)
    return out.reshape(b, s, hid)


# speedup vs baseline: 2.6789x; 1.0229x over previous
import functools

import jax
import jax.numpy as jnp
from jax.experimental import pallas as pl
from jax.experimental.pallas import tpu as pltpu

_NH, _HD = 16, 128
_CHUNK, _TOPK = 256, 4
_THETA = 10000.0
_SCALE = _HD ** -0.5
_NEG = -2e30


def _dot_t(a, b):
    return jax.lax.dot_general(a, b, (((1,), (1,)), ((), ())),
                               preferred_element_type=jnp.float32)


def _qk_kernel(h_ref, w_ref, cc_ref, sc_ref, o_ref):
    x = _dot_t(h_ref[...], w_ref[...])
    cc = cc_ref[...]
    sc = sc_ref[...]
    nheads = x.shape[1] // _HD
    for hh in range(nheads):
        xh = x[:, hh * _HD:(hh + 1) * _HD]
        xs = pltpu.roll(xh, _HD // 2, 1)
        o_ref[:, hh * _HD:(hh + 1) * _HD] = xh * cc + xs * sc


def _v_kernel(h_ref, w_ref, o_ref):
    o_ref[...] = jax.lax.dot_general(
        h_ref[...].astype(jnp.bfloat16), w_ref[...].astype(jnp.bfloat16),
        (((1,), (1,)), ((), ())),
        preferred_element_type=jnp.float32).astype(jnp.bfloat16)


def _attn_kernel(q_ref, k_ref, v_ref, o_ref, *, nchunks):
    c_sz = _CHUNK
    nh = q_ref.shape[1] // _HD
    hsls = [slice(hh * _HD, (hh + 1) * _HD) for hh in range(nh)]

    rowc = jax.lax.broadcasted_iota(jnp.int32, (nchunks, c_sz), 0)
    rowk = jax.lax.broadcasted_iota(jnp.int32, (c_sz, c_sz), 0)
    colq = jax.lax.broadcasted_iota(jnp.int32, (c_sz, c_sz), 1)
    tri_t = colq >= rowk
    ones_hd = jnp.ones((1, _HD), jnp.float32)
    ones_c = jnp.ones((1, c_sz), jnp.bfloat16)

    for hh, hsl in enumerate(hsls):
        kms, kmax_pre = [], []
        for c in range(nchunks):
            kc = k_ref[c * c_sz:(c + 1) * c_sz, hsl]
            kms.append(jnp.mean(kc, axis=0, keepdims=True))
            kn2 = jax.lax.dot_general(ones_hd, kc * kc, (((1,), (1,)), ((), ())),
                                      preferred_element_type=jnp.float32)
            kn = jnp.max(jnp.sqrt(kn2))
            kmax_pre.append(kn if c == 0 else jnp.maximum(kmax_pre[-1], kn))
        kmean = jnp.concatenate(kms, axis=0)

        for qt in range(nchunks):
            qtile = q_ref[qt * c_sz:(qt + 1) * c_sz, hsl]
            gt = _dot_t(kmean, qtile) * _SCALE
            g = jnp.where(rowc == qt, 1e9, gt)
            g = jnp.where(rowc > qt, -1e9, g)
            rank = jnp.zeros((nchunks, c_sz), jnp.float32)
            for d in range(1, nchunks):
                gs = jnp.concatenate([g[d:], g[:d]], axis=0)
                tie_lower = (rowc + d) >= nchunks
                beats = (gs > g) | ((gs == g) & tie_lower)
                rank = rank + beats.astype(jnp.float32)
            bias_t = jnp.where(rank < _TOPK, 0.0, _NEG)

            qn2 = jax.lax.dot_general(ones_hd, qtile * qtile,
                                      (((1,), (1,)), ((), ())),
                                      preferred_element_type=jnp.float32)
            bnd = jnp.sqrt(qn2) * (kmax_pre[qt] * _SCALE)

            qb = (qtile * _SCALE).astype(jnp.bfloat16)
            acc = jnp.zeros((_HD, c_sz), jnp.float32)
            l = jnp.zeros((1, c_sz), jnp.float32)
            for c in range(qt + 1):
                kc = k_ref[c * c_sz:(c + 1) * c_sz, hsl].astype(jnp.bfloat16)
                s = _dot_t(kc, qb)
                adj = (-bnd) if c == qt else (bias_t[c:c + 1, :] - bnd)
                p = jnp.exp(s + adj)
                if c == qt:
                    p = jnp.where(tri_t, p, 0.0)
                pb = p.astype(jnp.bfloat16)
                l = l + jax.lax.dot_general(ones_c, pb, (((1,), (0,)), ((), ())),
                                            preferred_element_type=jnp.float32)
                vc = v_ref[c * c_sz:(c + 1) * c_sz, hsl]
                acc = acc + jax.lax.dot_general(
                    vc, pb, (((0,), (0,)), ((), ())),
                    preferred_element_type=jnp.float32)
            o_ref[0, hsl, qt * c_sz:(qt + 1) * c_sz] = (
                acc * (1.0 / l)).astype(jnp.bfloat16)


def _proj_kernel(x_ref, w_ref, o_ref):
    o_ref[...] = jax.lax.dot_general(
        x_ref[0], w_ref[...].astype(jnp.bfloat16),
        (((0,), (1,)), ((), ())), preferred_element_type=jnp.float32)


def kernel(hidden_states, positions, Wqkv, Wo):
    b, s, hid = hidden_states.shape
    nchunks = s // _CHUNK
    rows = b * s
    hs = hidden_states.reshape(rows, hid)
    qk_heads = 2 * _NH

    inv = 1.0 / (_THETA ** (jnp.arange(0, _HD, 2, dtype=jnp.float32) / _HD))
    f = positions.astype(jnp.float32)[:, None] * inv[None, :]
    cos = jnp.cos(f)
    sin = jnp.sin(f)
    cc = jnp.concatenate([cos, cos], axis=1)
    sc = jnp.concatenate([-sin, sin], axis=1)

    params = pltpu.CompilerParams(
        dimension_semantics=("parallel", "parallel"))

    hpt = 4
    ncol = hpt * _HD
    qk = pl.pallas_call(
        _qk_kernel,
        grid=(b, qk_heads // hpt),
        in_specs=[
            pl.BlockSpec((s, hid), lambda i, j: (i, 0)),
            pl.BlockSpec((ncol, hid), lambda i, j: (j, 0)),
            pl.BlockSpec((s, _HD), lambda i, j: (0, 0)),
            pl.BlockSpec((s, _HD), lambda i, j: (0, 0)),
        ],
        out_specs=pl.BlockSpec((s, ncol), lambda i, j: (i, j)),
        out_shape=jax.ShapeDtypeStruct((rows, qk_heads * _HD), jnp.float32),
        compiler_params=params,
    )(hs, Wqkv, cc, sc)

    v = pl.pallas_call(
        _v_kernel,
        grid=(b, _NH // hpt),
        in_specs=[
            pl.BlockSpec((s, hid), lambda i, j: (i, 0)),
            pl.BlockSpec((ncol, hid), lambda i, j: (qk_heads // hpt + j, 0)),
        ],
        out_specs=pl.BlockSpec((s, ncol), lambda i, j: (i, j)),
        out_shape=jax.ShapeDtypeStruct((rows, _NH * _HD), jnp.bfloat16),
        compiler_params=params,
    )(hs, Wqkv)

    hpg = 2
    attn_t = pl.pallas_call(
        functools.partial(_attn_kernel, nchunks=nchunks),
        grid=(b, _NH // hpg),
        in_specs=[
            pl.BlockSpec((s, hpg * _HD), lambda bi, h: (bi, h)),
            pl.BlockSpec((s, hpg * _HD), lambda bi, h: (bi, _NH // hpg + h)),
            pl.BlockSpec((s, hpg * _HD), lambda bi, h: (bi, h)),
        ],
        out_specs=pl.BlockSpec((1, hpg * _HD, s), lambda bi, h: (bi, h, 0)),
        out_shape=jax.ShapeDtypeStruct((b, _NH * _HD, s), jnp.bfloat16),
        compiler_params=params,
    )(qk, qk, v)

    ocols = 512
    out = pl.pallas_call(
        _proj_kernel,
        grid=(b, hid // ocols),
        in_specs=[
            pl.BlockSpec((1, _NH * _HD, s), lambda i, j: (i, 0, 0)),
            pl.BlockSpec((ocols, _NH * _HD), lambda i, j: (j, 0)),
        ],
        out_specs=pl.BlockSpec((s, ocols), lambda i, j: (i, j)),
        out_shape=jax.ShapeDtypeStruct((rows, hid), jnp.float32),
        compiler_params=params,
    )(attn_t, Wo)
    return out.reshape(b, s, hid)
